# baseline (speedup 1.0000x reference)
import functools

import jax
import jax.numpy as jnp
from jax import lax
from jax.experimental import pallas as pl
from jax.experimental.pallas import tpu as pltpu
from jax.experimental.pallas import tpu_sc as plsc

_N_EXPERT = 100000
_D = 128
_A = 18
_B = 16384

_NC, _NS, _L = 2, 16, 16
_NW = _NC * _NS
_BPW = _B // _NW
_CHUNK = 128
_NCHUNK = _BPW // _CHUNK
_OHROWS = _N_EXPERT * _A // _L

_mesh = plsc.VectorSubcoreMesh(
    core_axis_name="c", subcore_axis_name="s", num_cores=_NC)


@functools.partial(
    pl.kernel,
    mesh=_mesh,
    compiler_params=pltpu.CompilerParams(
        needs_layout_passes=False, use_tc_tiling_on_sc=False),
    out_type=(
        jax.ShapeDtypeStruct((_B, _D), jnp.float32),
        jax.ShapeDtypeStruct((_B,), jnp.int32),
    ),
    scratch_types=[
        pltpu.VMEM((_NCHUNK, _CHUNK), jnp.int32),
        pltpu.VMEM((2 * _NCHUNK, _CHUNK), jnp.int32),
        pltpu.VMEM((_BPW, _D), jnp.float32),
        pltpu.VMEM((2, _BPW, _L), jnp.float32),
        pltpu.VMEM((_BPW,), jnp.int32),
        pltpu.SemaphoreType.DMA,
        pltpu.SemaphoreType.DMA,
    ],
)
def _gather_decode(states_hbm, oh16_hbm, idx_hbm, out_states, out_actions,
                   idx_v, smp_v, rows_v, win_v, act_v, sem_s, sem_a):
    wid = lax.axis_index("s") * _NC + lax.axis_index("c")
    base = wid * _BPW
    pltpu.sync_copy(idx_hbm.at[pl.ds(wid * _NCHUNK, _NCHUNK)], idx_v)

    for c in range(_NCHUNK):
        for o in range(_CHUNK // _L):
            idx16 = idx_v[c, pl.ds(o * _L, _L)]
            a = lax.shift_right_logical(idx16 * _A, 4)
            smp_v[c, pl.ds(o * _L, _L)] = a
            smp_v[_NCHUNK + c, pl.ds(o * _L, _L)] = a + 1

    copies = []
    for c in range(_NCHUNK):
        copies.append(pltpu.async_copy(
            states_hbm.at[idx_v.at[c]],
            rows_v.at[pl.ds(c * _CHUNK, _CHUNK)], sem_s))
        copies.append(pltpu.async_copy(
            oh16_hbm.at[smp_v.at[c]],
            win_v.at[0, pl.ds(c * _CHUNK, _CHUNK)], sem_a))
        copies.append(pltpu.async_copy(
            oh16_hbm.at[smp_v.at[_NCHUNK + c]],
            win_v.at[1, pl.ds(c * _CHUNK, _CHUNK)], sem_a))
    for cp in copies[1::3] + copies[2::3]:
        cp.wait()

    for c in range(_NCHUNK):
        for o in range(_CHUNK // _L):
            sl = pl.ds(c * _CHUNK + o * _L, _L)
            rows16 = (c * _CHUNK + o * _L) + lax.iota(jnp.int32, _L)
            idx16 = idx_v[c, pl.ds(o * _L, _L)]
            e = lax.bitwise_and(idx16 * _A, jnp.full((_L,), 15, jnp.int32))
            acc = jnp.zeros((_L,), jnp.float32)
            for j in range(1, _A):
                w = e + j
                sel = lax.shift_right_logical(w, 4)
                col = lax.bitwise_and(w, jnp.full((_L,), 15, jnp.int32))
                acc = acc + jnp.float32(j) * plsc.load_gather(
                    win_v, [sel, rows16, col])
            act_v[sl] = acc.astype(jnp.int32)

    pltpu.sync_copy(act_v, out_actions.at[pl.ds(base, _BPW)])

    for cp in copies[0::3]:
        cp.wait()
    pltpu.sync_copy(rows_v, out_states.at[pl.ds(base, _BPW)])


def kernel(expert_states, expert_actions, indices):
    idx2d = indices.astype(jnp.int32).reshape(_NW * _NCHUNK, _CHUNK)
    oh16 = expert_actions.reshape(_OHROWS, _L)
    states, actions = _gather_decode(expert_states, oh16, idx2d)
    return (states, actions)
